# baseline (speedup 1.0000x reference)
import functools

import jax
import jax.numpy as jnp
from jax import lax
from jax.experimental import pallas as pl
from jax.experimental.pallas import tpu as pltpu
from jax.experimental.pallas import tpu_sc as plsc

_NC = 2
_NS = 16
_NW = _NC * _NS
_IDX_CHUNK = 128


def _gather_sum(contexts, emb_table):
    B, L = contexts.shape
    _, D = emb_table.shape
    b_per_w = B // _NW
    n_idx = b_per_w * L
    n_ch = n_idx // _IDX_CHUNK
    assert B % _NW == 0 and n_idx % _IDX_CHUNK == 0

    ctx_flat = contexts.reshape(-1)

    mesh = plsc.VectorSubcoreMesh(core_axis_name="c", subcore_axis_name="s")

    @functools.partial(
        pl.kernel,
        mesh=mesh,
        out_type=jax.ShapeDtypeStruct((B, D), jnp.float32),
        scratch_types=[
            pltpu.VMEM((n_idx,), jnp.int32),
            pltpu.VMEM((n_idx, D), jnp.float32),
            pltpu.VMEM((b_per_w, D), jnp.float32),
            pltpu.SemaphoreType.DMA,
        ],
        compiler_params=pltpu.CompilerParams(use_tc_tiling_on_sc=False),
    )
    def sc_kernel(ctx_hbm, table_hbm, out_hbm, idx_v, rows_v, acc_v, sem):
        wid = lax.axis_index("s") * _NC + lax.axis_index("c")
        pltpu.sync_copy(ctx_hbm.at[pl.ds(wid * n_idx, n_idx)], idx_v)
        copies = []
        for c in range(n_ch):
            copies.append(
                pltpu.async_copy(
                    table_hbm.at[idx_v.at[pl.ds(c * _IDX_CHUNK, _IDX_CHUNK)]],
                    rows_v.at[pl.ds(c * _IDX_CHUNK, _IDX_CHUNK)],
                    sem,
                )
            )
        for cp in copies:
            cp.wait()

        def body(b, _):
            acc = rows_v[b * L, :]
            for l in range(1, L):
                acc = acc + rows_v[b * L + l, :]
            acc_v[b, :] = acc
            return 0

        lax.fori_loop(0, b_per_w, body, 0)
        pltpu.sync_copy(acc_v, out_hbm.at[pl.ds(wid * b_per_w, b_per_w)])

    return sc_kernel(ctx_flat, emb_table)


def _proj_logsoftmax(x, W, b, block_v=2048):
    B, D = x.shape
    V = W.shape[0]
    nbv = pl.cdiv(V, block_v)
    b2d = b.reshape(1, V)

    def tc_kernel(x_ref, w_ref, b_ref, out_ref, m_scr, s_scr, lse_scr):
        phase = pl.program_id(0)
        j = pl.program_id(1)
        logits = (
            lax.dot_general(
                x_ref[...], w_ref[...],
                (((1,), (1,)), ((), ())),
                preferred_element_type=jnp.float32,
            )
            + b_ref[...]
        )
        col = j * block_v + lax.broadcasted_iota(jnp.int32, (B, block_v), 1)
        logits = jnp.where(col < V, logits, -jnp.inf)

        @pl.when((phase == 0) & (j == 0))
        def _():
            m_scr[...] = jnp.full_like(m_scr, -jnp.inf)
            s_scr[...] = jnp.zeros_like(s_scr)

        @pl.when(phase == 0)
        def _():
            m_old = m_scr[...]
            m_new = jnp.maximum(m_old, jnp.max(logits, axis=1, keepdims=True))
            s_scr[...] = s_scr[...] * jnp.exp(m_old - m_new) + jnp.sum(
                jnp.exp(logits - m_new), axis=1, keepdims=True
            )
            m_scr[...] = m_new

        @pl.when((phase == 0) & (j == nbv - 1))
        def _():
            lse_scr[...] = m_scr[...] + jnp.log(s_scr[...])

        @pl.when(phase == 1)
        def _():
            out_ref[...] = logits - lse_scr[...]

    return pl.pallas_call(
        tc_kernel,
        grid=(2, nbv),
        in_specs=[
            pl.BlockSpec((B, D), lambda p, j: (0, 0)),
            pl.BlockSpec((block_v, D), lambda p, j: (j, 0)),
            pl.BlockSpec((1, block_v), lambda p, j: (0, j)),
        ],
        out_specs=pl.BlockSpec((B, block_v), lambda p, j: (0, j * p)),
        out_shape=jax.ShapeDtypeStruct((B, V), jnp.float32),
        scratch_shapes=[
            pltpu.VMEM((B, 1), jnp.float32),
            pltpu.VMEM((B, 1), jnp.float32),
            pltpu.VMEM((B, 1), jnp.float32),
        ],
    )(x, W, b2d)


def kernel(contexts, emb_table, W, b):
    add_embeds = _gather_sum(contexts, emb_table)
    return _proj_logsoftmax(add_embeds, W, b)
